# Initial kernel scaffold; baseline (speedup 1.0000x reference)
import jax
import jax.numpy as jnp
from jax.experimental import pallas as pl


def kernel(x, token_table, pos_table):
    raise NotImplementedError("write your pallas kernel here")


# speedup vs baseline: 7.0160x; 7.0160x over previous
import functools

import jax
import jax.numpy as jnp
from jax import lax
from jax.experimental import pallas as pl
from jax.experimental.pallas import tpu as pltpu
from jax.experimental.pallas import tpu_sc as plsc

NUM_WORKERS = 32
CHUNK = 1600


def _sc_gather(table, idx):
    n, d = idx.shape[0], table.shape[1]
    per_w = n // NUM_WORKERS
    n_chunks = per_w // CHUNK
    mesh = plsc.VectorSubcoreMesh(core_axis_name="c", subcore_axis_name="s")

    @functools.partial(
        pl.kernel,
        mesh=mesh,
        out_type=jax.ShapeDtypeStruct((n, d), jnp.float32),
        compiler_params=pltpu.CompilerParams(use_tc_tiling_on_sc=False),
        scratch_types=[
            pltpu.VMEM((CHUNK,), jnp.int32),
            pltpu.VMEM((CHUNK, d), jnp.float32),
            pltpu.SemaphoreType.DMA,
        ],
    )
    def gather_kernel(table_hbm, idx_hbm, out_hbm, idx_v, rows_v, sem):
        wid = lax.axis_index("s") * 2 + lax.axis_index("c")
        base = wid * per_w

        @pl.loop(0, n_chunks)
        def _(ci):
            off = base + ci * CHUNK
            pltpu.sync_copy(idx_hbm.at[pl.ds(off, CHUNK)], idx_v)
            pltpu.async_copy(table_hbm.at[idx_v], rows_v, sem).wait()
            pltpu.sync_copy(rows_v, out_hbm.at[pl.ds(off, CHUNK)])

    return gather_kernel(table, idx)


def _tc_add_body(tok_ref, pos_ref, o_ref):
    o_ref[...] = tok_ref[...] + pos_ref[...]


def _tc_add(tok2d, pos2d):
    b, w = tok2d.shape
    rows = 128
    return pl.pallas_call(
        _tc_add_body,
        grid=(b // rows,),
        in_specs=[
            pl.BlockSpec((rows, w), lambda i: (i, 0)),
            pl.BlockSpec((1, w), lambda i: (0, 0)),
        ],
        out_specs=pl.BlockSpec((rows, w), lambda i: (i, 0)),
        out_shape=jax.ShapeDtypeStruct((b, w), jnp.float32),
    )(tok2d, pos2d)


def kernel(x, token_table, pos_table):
    b, maxlen = x.shape
    d = token_table.shape[1]
    xf = x.reshape(-1).astype(jnp.int32)
    tok = _sc_gather(token_table, xf)
    out2d = _tc_add(tok.reshape(b, maxlen * d),
                    pos_table.reshape(1, maxlen * d))
    return out2d.reshape(b, maxlen, d)


# speedup vs baseline: 7.2283x; 1.0303x over previous
import functools

import jax
import jax.numpy as jnp
from jax import lax
from jax.experimental import pallas as pl
from jax.experimental.pallas import tpu as pltpu
from jax.experimental.pallas import tpu_sc as plsc

NUM_WORKERS = 32
CHUNK = 1600


def _sc_gather(table, idx):
    n, d = idx.shape[0], table.shape[1]
    per_w = n // NUM_WORKERS
    n_chunks = per_w // CHUNK
    mesh = plsc.VectorSubcoreMesh(core_axis_name="c", subcore_axis_name="s")

    @functools.partial(
        pl.kernel,
        mesh=mesh,
        out_type=jax.ShapeDtypeStruct((n, d), jnp.float32),
        compiler_params=pltpu.CompilerParams(use_tc_tiling_on_sc=False),
        scratch_types=[
            pltpu.VMEM((CHUNK,), jnp.int32),
            pltpu.VMEM((CHUNK,), jnp.int32),
            pltpu.VMEM((CHUNK, d), jnp.float32),
            pltpu.VMEM((CHUNK, d), jnp.float32),
            pltpu.SemaphoreType.DMA,
            pltpu.SemaphoreType.DMA,
            pltpu.SemaphoreType.DMA,
            pltpu.SemaphoreType.DMA,
        ],
    )
    def gather_kernel(table_hbm, idx_hbm, out_hbm,
                      idx0, idx1, rows0, rows1, g0, g1, w0, w1):
        wid = lax.axis_index("s") * 2 + lax.axis_index("c")
        base = wid * per_w
        idx_v = (idx0, idx1)
        rows_v = (rows0, rows1)
        gsem = (g0, g1)
        wsem = (w0, w1)

        def start_gather(ci, b):
            off = base + ci * CHUNK
            pltpu.sync_copy(idx_hbm.at[pl.ds(off, CHUNK)], idx_v[b])
            pltpu.async_copy(table_hbm.at[idx_v[b]], rows_v[b], gsem[b])

        def wait_gather(b):
            pltpu.make_async_copy(table_hbm.at[idx_v[b]], rows_v[b],
                                  gsem[b]).wait()

        def start_writeback(ci, b):
            off = base + ci * CHUNK
            pltpu.async_copy(rows_v[b], out_hbm.at[pl.ds(off, CHUNK)], wsem[b])

        def wait_writeback(ci, b):
            off = base + ci * CHUNK
            pltpu.make_async_copy(rows_v[b], out_hbm.at[pl.ds(off, CHUNK)],
                                  wsem[b]).wait()

        start_gather(0, 0)

        @pl.loop(0, n_chunks, step=2)
        def _(ci):
            for b in range(2):
                cur = ci + b
                nxt = cur + 1

                @pl.when(nxt < n_chunks)
                def _():
                    @pl.when(nxt >= 2)
                    def _():
                        wait_writeback(nxt - 2, 1 - b)
                    start_gather(nxt, 1 - b)

                wait_gather(b)
                start_writeback(cur, b)

        wait_writeback(n_chunks - 2, 0)
        wait_writeback(n_chunks - 1, 1)

    return gather_kernel(table, idx)


def _tc_add_body(tok_ref, pos_ref, o_ref):
    o_ref[...] = tok_ref[...] + pos_ref[...]


def _tc_add(tok2d, pos2d):
    b, w = tok2d.shape
    rows = 128
    return pl.pallas_call(
        _tc_add_body,
        grid=(b // rows,),
        in_specs=[
            pl.BlockSpec((rows, w), lambda i: (i, 0)),
            pl.BlockSpec((1, w), lambda i: (0, 0)),
        ],
        out_specs=pl.BlockSpec((rows, w), lambda i: (i, 0)),
        out_shape=jax.ShapeDtypeStruct((b, w), jnp.float32),
    )(tok2d, pos2d)


def kernel(x, token_table, pos_table):
    b, maxlen = x.shape
    d = token_table.shape[1]
    xf = x.reshape(-1).astype(jnp.int32)
    tok = _sc_gather(token_table, xf)
    out2d = _tc_add(tok.reshape(b, maxlen * d),
                    pos_table.reshape(1, maxlen * d))
    return out2d.reshape(b, maxlen, d)
